# Initial kernel scaffold; baseline (speedup 1.0000x reference)
import jax
import jax.numpy as jnp
from jax.experimental import pallas as pl


def kernel(x1, x2, x3, x4, edge_index1, edge_index2, edge_index3, edge_index4, t_data, Wg1, bg1, Wf1, bf1, Wg2, bg2, Wf2, bf2, Wg3, bg3, Wf3, bf3, Wg4, bg4, Wf4, bf4, Ws, bs, Wt, bt, W1, b1, W2, b2):
    raise NotImplementedError("write your pallas kernel here")


# speedup vs baseline: 4.8860x; 4.8860x over previous
import jax
import jax.numpy as jnp
from jax import lax
from jax.experimental import pallas as pl
from jax.experimental.pallas import tpu as pltpu
from jax.experimental.pallas import tpu_sc as plsc

F32 = jnp.float32
I32 = jnp.int32

N_NODES = 85
NP = 96
ROWS = 24
EPG = 1024
EPT = 256
NG = 4
NT = 16
FPT = 480


def _rsqrt_nr(d):
    y = jnp.full((16,), 0.5, F32)
    for thr in (4.0, 16.0, 64.0, 256.0, 1024.0):
        y = jnp.where(d >= thr, 0.5 * y, y)
    for _ in range(6):
        y = y * (1.5 - 0.5 * d * y * y)
    return y


def _hsum(v):
    s = v[0]
    for l in range(1, 16):
        s = s + v[l]
    return s


def _body(x_hbm, src_hbm, dst_hbm, wg_hbm, bg_hbm, wft_hbm, bfl_hbm,
          ws_hbm, w1_hbm, td_hbm, wt_hbm, bt_hbm, cst_hbm, ones_hbm,
          out_hbm,
          deg_s, h_s, out_s, part_s,
          xv, wgv, srcv, dstv, srcidx, dstidx, degv, dinvv, hv, msgv,
          outv, zero16, onesv, flatv, w0v, w1gv, wftv, wsv, w1v, bgv, tv,
          wtv, btv, bflv, cstv, partv, pallv, outb, sem):
    s = lax.axis_index("s")
    g = s // 4
    q = s % 4
    rbase = ROWS * q
    grow = NP * g + rbase

    iota = lax.iota(I32, 16)
    zf = jnp.zeros((16,), F32)
    e0 = jnp.where(iota == 0, 1.0, 0.0).astype(F32)

    copies = [
        pltpu.async_copy(x_hbm.at[pl.ds(grow, ROWS)], xv, sem),
        pltpu.async_copy(wg_hbm.at[pl.ds(32 * g, 32)], wgv, sem),
        pltpu.async_copy(src_hbm.at[pl.ds(EPG * g + EPT * q, EPT)], srcv, sem),
        pltpu.async_copy(dst_hbm.at[pl.ds(EPG * g + EPT * q, EPT)], dstv, sem),
        pltpu.async_copy(ones_hbm, onesv, sem),
        pltpu.async_copy(wft_hbm.at[pl.ds(8 * g, 8)], wftv, sem),
        pltpu.async_copy(bg_hbm.at[pl.ds(32 * g, 32)], bgv, sem),
        pltpu.async_copy(ws_hbm, wsv, sem),
        pltpu.async_copy(w1_hbm, w1v, sem),
        pltpu.async_copy(td_hbm, tv, sem),
        pltpu.async_copy(wt_hbm, wtv, sem),
        pltpu.async_copy(bt_hbm, btv, sem),
        pltpu.async_copy(bfl_hbm, bflv, sem),
        pltpu.async_copy(cst_hbm, cstv, sem),
    ]

    for i in range(ROWS):
        outv[i, 0:16] = zf
        outv[i, 16:32] = zf
        zero16[i, :] = zf

    for c in copies:
        c.wait()

    @pl.when(q == 0)
    def _():
        for r in range(4):
            pltpu.sync_copy(outv, out_s.at[pl.ds(NP * g + ROWS * r, ROWS)])
            pltpu.sync_copy(zero16, deg_s.at[pl.ds(NP * g + ROWS * r, ROWS)])

    def idx_step(j, _):
        srcidx[j // 8, pl.ds(16 * (j % 8), 16)] = (
            srcv[pl.ds(16 * j, 16)] + NP * g)
        dstidx[j // 8, pl.ds(16 * (j % 8), 16)] = (
            dstv[pl.ds(16 * j, 16)] + NP * g)
        return 0

    for j in range(EPT // 16):
        idx_step(j, 0)

    plsc.subcore_barrier()

    for t in range(NT):
        @pl.when(s == t)
        def _():
            for j in range(2):
                pltpu.sync_copy(onesv.at[pl.ds(128 * j, 128)],
                                deg_s.at[dstidx.at[j]], add=True)
        plsc.subcore_barrier()

    def mm_step(i, _):
        a1 = zf
        a2 = zf
        xr1 = xv[i, 0:16]
        xr2 = xv[i, 16:32]
        for k in range(32):
            xs = xr1[k] if k < 16 else xr2[k - 16]
            a1 = a1 + xs * wgv[k, 0:16]
            a2 = a2 + xs * wgv[k, 16:32]
        hv[i, 0:16] = a1
        hv[i, 16:32] = a2
        return 0

    lax.fori_loop(0, ROWS, mm_step, 0)

    plsc.subcore_barrier()

    pltpu.sync_copy(deg_s.at[pl.ds(grow, ROWS)], degv)

    def dinv_step(i, _):
        dinvv[i, :] = _rsqrt_nr(degv[i, :] + 1.0)
        hv[i, 0:16] = hv[i, 0:16] * dinvv[i, 0:16]
        hv[i, 16:32] = hv[i, 16:32] * dinvv[i, 0:16]
        return 0

    lax.fori_loop(0, ROWS, dinv_step, 0)
    pltpu.sync_copy(hv, h_s.at[pl.ds(grow, ROWS)])

    plsc.subcore_barrier()

    for j in range(2):
        pltpu.sync_copy(h_s.at[srcidx.at[j]],
                        msgv.at[pl.ds(128 * j, 128)])
    for t in range(NT):
        @pl.when(s == t)
        def _():
            for j in range(2):
                pltpu.sync_copy(msgv.at[pl.ds(128 * j, 128)],
                                out_s.at[dstidx.at[j]], add=True)
        plsc.subcore_barrier()

    plsc.subcore_barrier()

    u1 = zf
    u2 = zf
    wsg = wsv[pl.ds(7 * g, 16)]
    for c in range(7):
        wssc = wsg[c]
        u1 = u1 + wssc * wftv[c, 0:16]
        u2 = u2 + wssc * wftv[c, 16:32]

    pltpu.sync_copy(out_s.at[pl.ds(grow, ROWS)], outv)

    def flat_step(i, _):
        dsc = dinvv[i, 0:16]
        o1 = jnp.maximum((outv[i, 0:16] + hv[i, 0:16]) * dsc + bgv[0:16], 0.0)
        o2 = jnp.maximum((outv[i, 16:32] + hv[i, 16:32]) * dsc + bgv[16:32],
                         0.0)
        flatv[pl.ds(20 * i + 16, 16)] = o2
        flatv[pl.ds(20 * i, 16)] = o1
        return 0

    lax.fori_loop(0, ROWS, flat_step, 0)

    for m in range(FPT // 16 + 4):
        w0v[pl.ds(16 * m, 16)] = zf
        w1gv[pl.ds(16 * m, 16)] = zf

    fbase = FPT * q
    r20 = fbase // 17
    r2hi = (fbase + FPT - 1) // 17 + 1

    def wb_step(r2, _):
        pz = r2 // 50
        w1s = w1v[pl.ds(r2 - 50 * pz, 16)][0]
        off = 17 * r2 - fbase + 16
        r2v = lax.convert_element_type(iota * 0 + r2, F32)
        m0 = jnp.where(r2v < 50.0, 1.0, 0.0)
        m1 = jnp.where(r2v < 100.0, 1.0, 0.0) - m0
        main = u1 * w1s
        tail = e0 * (u2 * w1s)
        w0v[pl.ds(off + 16, 16)] = tail * m0
        w0v[pl.ds(off, 16)] = main * m0
        w1gv[pl.ds(off + 16, 16)] = tail * m1
        w1gv[pl.ds(off, 16)] = main * m1
        return 0

    lax.fori_loop(r20, r2hi, wb_step, 0)

    acc0 = zf
    acc1 = zf
    for m in range(FPT // 16):
        fv = flatv[pl.ds(16 * m, 16)]
        acc0 = acc0 + fv * w0v[pl.ds(16 * m + 16, 16)]
        acc1 = acc1 + fv * w1gv[pl.ds(16 * m + 16, 16)]

    partv[0, 0:16] = acc0
    partv[0, 16:32] = acc1
    pltpu.sync_copy(partv, part_s.at[pl.ds(s, 1)])

    plsc.subcore_barrier()

    @pl.when(s == 0)
    def _():
        pltpu.sync_copy(part_s, pallv)
        av0 = zf
        av1 = zf
        for t in range(NT):
            av0 = av0 + pallv[t, 0:16]
            av1 = av1 + pallv[t, 16:32]
        sz0 = _hsum(av0)
        sz1 = _hsum(av1)
        cb = cstv[:]

        cconst = cb[0]
        for gg in range(4):
            wsg0 = wsv[pl.ds(7 * gg, 16)]
            bfg = bflv[pl.ds(8 * gg, 16)]
            cconst = cconst + _hsum(
                jnp.where(iota < 7, bfg * wsg0, 0.0))
        sw1 = (_hsum(w1v[0:16] + w1v[16:32] + w1v[32:48])
               + _hsum(jnp.where(iota < 2, w1v[48:64], 0.0)))

        xa = btv[0:16]
        t1 = tv[0:16]
        t2 = tv[16:32]
        for k in range(24):
            ts = t1[k] if k < 16 else t2[k - 16]
            xa = xa + ts * wtv[k, 0:16]

        w1hi = w1v[48:64]
        base = cb[1] + cconst * sw1
        z0 = base + sz0
        z1 = base + sz1
        for j in range(7):
            z0 = z0 + xa[j] * w1hi[2 + j]
            z1 = z1 + xa[7 + j] * w1hi[2 + j]
        fin = cb[2] + z0 * cb[3] + z1 * cb[4]
        outb[:] = e0 * fin
        pltpu.sync_copy(outb, out_hbm)


@jax.jit
def kernel(x1, x2, x3, x4, edge_index1, edge_index2, edge_index3,
           edge_index4, t_data, Wg1, bg1, Wf1, bf1, Wg2, bg2, Wf2, bf2,
           Wg3, bg3, Wf3, bf3, Wg4, bg4, Wf4, bf4,
           Ws, bs, Wt, bt, W1, b1, W2, b2):
    xs = (x1, x2, x3, x4)
    eis = (edge_index1, edge_index2, edge_index3, edge_index4)
    wgs = (Wg1, Wg2, Wg3, Wg4)
    bgs = (bg1, bg2, bg3, bg4)
    wfs = (Wf1, Wf2, Wf3, Wf4)
    bfs = (bf1, bf2, bf3, bf4)

    X = jnp.concatenate(
        [jnp.pad(x, ((0, NP - N_NODES), (0, 0))) for x in xs], axis=0)
    SRC = jnp.concatenate(
        [jnp.pad(ei[0], (0, EPG - ei.shape[1]), constant_values=NP - 1)
         for ei in eis])
    DST = jnp.concatenate(
        [jnp.pad(ei[1], (0, EPG - ei.shape[1]), constant_values=NP - 1)
         for ei in eis])
    WG = jnp.concatenate(
        [jnp.pad(w, ((0, 0), (0, 12))) for w in wgs], axis=0)
    BG = jnp.concatenate([jnp.pad(b, (0, 12)) for b in bgs])
    WFT = jnp.concatenate(
        [jnp.pad(w.T, ((0, 1), (0, 15))) for w in wfs], axis=0)
    BFL = jnp.pad(jnp.concatenate([jnp.pad(b, (0, 1)) for b in bfs]),
                  (0, 16))
    WSF = jnp.pad(Ws[:, 0], (0, 20))
    W1F = jnp.pad(W1[:, 0], (0, 23))
    TD = jnp.pad(t_data[0], (0, 8))
    WT = jnp.pad(Wt, ((0, 0), (0, 2)))
    BT = jnp.pad(bt, (0, 2))
    CST = jnp.concatenate(
        [bs, b1, b2, W2[:, 0], jnp.zeros((11,), F32)])
    ONES = jnp.ones((EPT, 16), F32)

    mesh = plsc.VectorSubcoreMesh(
        core_axis_name="c", subcore_axis_name="s", num_cores=1)
    run = pl.kernel(
        _body,
        out_type=jax.ShapeDtypeStruct((16,), F32),
        mesh=mesh,
        scratch_types=[
            pltpu.VMEM_SHARED((NG * NP, 16), F32),
            pltpu.VMEM_SHARED((NG * NP, 32), F32),
            pltpu.VMEM_SHARED((NG * NP, 32), F32),
            pltpu.VMEM_SHARED((NT, 32), F32),
            pltpu.VMEM((ROWS, 32), F32),
            pltpu.VMEM((32, 32), F32),
            pltpu.VMEM((EPT,), I32),
            pltpu.VMEM((EPT,), I32),
            pltpu.VMEM((2, 128), I32),
            pltpu.VMEM((2, 128), I32),
            pltpu.VMEM((ROWS, 16), F32),
            pltpu.VMEM((ROWS, 16), F32),
            pltpu.VMEM((ROWS, 32), F32),
            pltpu.VMEM((EPT, 32), F32),
            pltpu.VMEM((ROWS, 32), F32),
            pltpu.VMEM((ROWS, 16), F32),
            pltpu.VMEM((EPT, 16), F32),
            pltpu.VMEM((512,), F32),
            pltpu.VMEM((FPT + 64,), F32),
            pltpu.VMEM((FPT + 64,), F32),
            pltpu.VMEM((8, 32), F32),
            pltpu.VMEM((48,), F32),
            pltpu.VMEM((80,), F32),
            pltpu.VMEM((32,), F32),
            pltpu.VMEM((32,), F32),
            pltpu.VMEM((24, 16), F32),
            pltpu.VMEM((16,), F32),
            pltpu.VMEM((48,), F32),
            pltpu.VMEM((16,), F32),
            pltpu.VMEM((1, 32), F32),
            pltpu.VMEM((NT, 32), F32),
            pltpu.VMEM((16,), F32),
            pltpu.SemaphoreType.DMA,
        ],
    )
    out16 = run(X, SRC, DST, WG, BG, WFT, BFL, WSF, W1F, TD, WT, BT, CST,
                ONES)
    return out16[0:1]


# speedup vs baseline: 5.5984x; 1.1458x over previous
import jax
import jax.numpy as jnp
from jax import lax
from jax.experimental import pallas as pl
from jax.experimental.pallas import tpu as pltpu
from jax.experimental.pallas import tpu_sc as plsc

F32 = jnp.float32
I32 = jnp.int32

N_NODES = 85
NP = 96
ROWS = 24
EPG = 1024
EPT = 256
NG = 4
NT = 16
FPT = 480


def _rsqrt_nr(d):
    y = jnp.full((16,), 0.5, F32)
    for thr in (4.0, 16.0, 64.0, 256.0, 1024.0):
        y = jnp.where(d >= thr, 0.5 * y, y)
    for _ in range(6):
        y = y * (1.5 - 0.5 * d * y * y)
    return y


def _hsum(v):
    s = v[0]
    for l in range(1, 16):
        s = s + v[l]
    return s


def _body(x_hbm, src_hbm, dst_hbm, wg_hbm, bg_hbm, wft_hbm, bfl_hbm,
          ws_hbm, w1_hbm, td_hbm, wt_hbm, bt_hbm, cst_hbm,
          out_hbm,
          h_s, out_s, part_s,
          xv, wgv, srcv, dstfull, srcidx, dstidx, hv, msgv,
          outv, dstfv, flatv, w0v, w1gv, wftv, wsv, w1v, bgv, tv,
          wtv, btv, bflv, cstv, partv, pallv, outb, sem):
    s = lax.axis_index("s")
    g = s // 4
    q = s % 4
    rbase = ROWS * q
    grow = NP * g + rbase

    iota = lax.iota(I32, 16)
    zf = jnp.zeros((16,), F32)
    e0 = jnp.where(iota == 0, 1.0, 0.0).astype(F32)

    copies = [
        pltpu.async_copy(x_hbm.at[pl.ds(grow, ROWS)], xv, sem),
        pltpu.async_copy(wg_hbm.at[pl.ds(32 * g, 32)], wgv, sem),
        pltpu.async_copy(src_hbm.at[pl.ds(EPG * g + EPT * q, EPT)], srcv, sem),
        pltpu.async_copy(dst_hbm.at[pl.ds(EPG * g, EPG)], dstfull, sem),
        pltpu.async_copy(wft_hbm.at[pl.ds(8 * g, 8)], wftv, sem),
        pltpu.async_copy(bg_hbm.at[pl.ds(32 * g, 32)], bgv, sem),
        pltpu.async_copy(ws_hbm, wsv, sem),
        pltpu.async_copy(w1_hbm, w1v, sem),
        pltpu.async_copy(td_hbm, tv, sem),
        pltpu.async_copy(wt_hbm, wtv, sem),
        pltpu.async_copy(bt_hbm, btv, sem),
        pltpu.async_copy(bfl_hbm, bflv, sem),
        pltpu.async_copy(cst_hbm, cstv, sem),
    ]

    for i in range(ROWS):
        outv[i, 0:16] = zf
        outv[i, 16:32] = zf

    for c in copies:
        c.wait()

    @pl.when(q == 0)
    def _():
        for r in range(4):
            pltpu.sync_copy(outv, out_s.at[pl.ds(NP * g + ROWS * r, ROWS)])

    def dstf_step(j, _):
        dstfv[pl.ds(16 * j, 16)] = lax.convert_element_type(
            dstfull[pl.ds(16 * j, 16)], F32)
        return 0

    lax.fori_loop(0, EPG // 16, dstf_step, 0)

    def idx_step(j, _):
        srcidx[j // 8, pl.ds(16 * (j % 8), 16)] = (
            srcv[pl.ds(16 * j, 16)] + NP * g)
        dstidx[j // 8, pl.ds(16 * (j % 8), 16)] = (
            dstfull[pl.ds(EPT * q + 16 * j, 16)] + NP * g)
        return 0

    for j in range(EPT // 16):
        idx_step(j, 0)

    plsc.subcore_barrier()

    nv0 = lax.convert_element_type(iota + rbase, F32)
    nv1 = lax.convert_element_type(iota + (rbase + 16), F32)

    def deg_step(j, carry):
        c0, c1 = carry
        ev = dstfv[pl.ds(16 * j, 16)]
        for l in range(16):
            d = ev[l]
            c0 = c0 + jnp.where(nv0 == d, 1.0, 0.0)
            c1 = c1 + jnp.where(nv1 == d, 1.0, 0.0)
        return c0, c1

    c0, c1 = lax.fori_loop(0, EPG // 16, deg_step, (zf, zf))
    dv0 = _rsqrt_nr(c0 + 1.0)
    dv1 = _rsqrt_nr(c1 + 1.0)

    def mm_step(i, _):
        a1 = zf
        a2 = zf
        xr1 = xv[i, 0:16]
        xr2 = xv[i, 16:32]
        for k in range(32):
            xs = xr1[k] if k < 16 else xr2[k - 16]
            a1 = a1 + xs * wgv[k, 0:16]
            a2 = a2 + xs * wgv[k, 16:32]
        hv[i, 0:16] = a1
        hv[i, 16:32] = a2
        return 0

    lax.fori_loop(0, ROWS, mm_step, 0)

    for i in range(ROWS):
        dsc = dv0[i] if i < 16 else dv1[i - 16]
        hv[i, 0:16] = hv[i, 0:16] * dsc
        hv[i, 16:32] = hv[i, 16:32] * dsc
    pltpu.sync_copy(hv, h_s.at[pl.ds(grow, ROWS)])

    plsc.subcore_barrier()

    for j in range(2):
        pltpu.sync_copy(h_s.at[srcidx.at[j]],
                        msgv.at[pl.ds(128 * j, 128)])
    for t in range(NT):
        @pl.when(s == t)
        def _():
            for j in range(2):
                pltpu.sync_copy(msgv.at[pl.ds(128 * j, 128)],
                                out_s.at[dstidx.at[j]], add=True)
        plsc.subcore_barrier()

    plsc.subcore_barrier()

    u1 = zf
    u2 = zf
    wsg = wsv[pl.ds(7 * g, 16)]
    for c in range(7):
        wssc = wsg[c]
        u1 = u1 + wssc * wftv[c, 0:16]
        u2 = u2 + wssc * wftv[c, 16:32]

    pltpu.sync_copy(out_s.at[pl.ds(grow, ROWS)], outv)

    for i in range(ROWS):
        dsc = dv0[i] if i < 16 else dv1[i - 16]
        o1 = jnp.maximum((outv[i, 0:16] + hv[i, 0:16]) * dsc + bgv[0:16], 0.0)
        o2 = jnp.maximum((outv[i, 16:32] + hv[i, 16:32]) * dsc + bgv[16:32],
                         0.0)
        flatv[pl.ds(20 * i + 16, 16)] = o2
        flatv[pl.ds(20 * i, 16)] = o1

    for m in range(FPT // 16 + 4):
        w0v[pl.ds(16 * m, 16)] = zf
        w1gv[pl.ds(16 * m, 16)] = zf

    fbase = FPT * q
    r20 = fbase // 17
    r2hi = (fbase + FPT - 1) // 17 + 1

    def wb_step(r2, _):
        pz = r2 // 50
        w1s = w1v[pl.ds(r2 - 50 * pz, 16)][0]
        off = 17 * r2 - fbase + 16
        r2v = lax.convert_element_type(iota * 0 + r2, F32)
        m0 = jnp.where(r2v < 50.0, 1.0, 0.0)
        m1 = jnp.where(r2v < 100.0, 1.0, 0.0) - m0
        main = u1 * w1s
        tail = e0 * (u2 * w1s)
        w0v[pl.ds(off + 16, 16)] = tail * m0
        w0v[pl.ds(off, 16)] = main * m0
        w1gv[pl.ds(off + 16, 16)] = tail * m1
        w1gv[pl.ds(off, 16)] = main * m1
        return 0

    lax.fori_loop(r20, r2hi, wb_step, 0)

    acc0 = zf
    acc1 = zf
    for m in range(FPT // 16):
        fv = flatv[pl.ds(16 * m, 16)]
        acc0 = acc0 + fv * w0v[pl.ds(16 * m + 16, 16)]
        acc1 = acc1 + fv * w1gv[pl.ds(16 * m + 16, 16)]

    partv[0, 0:16] = acc0
    partv[0, 16:32] = acc1
    pltpu.sync_copy(partv, part_s.at[pl.ds(s, 1)])

    plsc.subcore_barrier()

    @pl.when(s == 0)
    def _():
        pltpu.sync_copy(part_s, pallv)
        av0 = zf
        av1 = zf
        for t in range(NT):
            av0 = av0 + pallv[t, 0:16]
            av1 = av1 + pallv[t, 16:32]
        sz0 = _hsum(av0)
        sz1 = _hsum(av1)
        cb = cstv[:]

        cconst = cb[0]
        for gg in range(4):
            wsg0 = wsv[pl.ds(7 * gg, 16)]
            bfg = bflv[pl.ds(8 * gg, 16)]
            cconst = cconst + _hsum(
                jnp.where(iota < 7, bfg * wsg0, 0.0))
        sw1 = (_hsum(w1v[0:16] + w1v[16:32] + w1v[32:48])
               + _hsum(jnp.where(iota < 2, w1v[48:64], 0.0)))

        xa = btv[0:16]
        t1 = tv[0:16]
        t2 = tv[16:32]
        for k in range(24):
            ts = t1[k] if k < 16 else t2[k - 16]
            xa = xa + ts * wtv[k, 0:16]

        w1hi = w1v[48:64]
        base = cb[1] + cconst * sw1
        z0 = base + sz0
        z1 = base + sz1
        for j in range(7):
            z0 = z0 + xa[j] * w1hi[2 + j]
            z1 = z1 + xa[7 + j] * w1hi[2 + j]
        fin = cb[2] + z0 * cb[3] + z1 * cb[4]
        outb[:] = e0 * fin
        pltpu.sync_copy(outb, out_hbm)


@jax.jit
def kernel(x1, x2, x3, x4, edge_index1, edge_index2, edge_index3,
           edge_index4, t_data, Wg1, bg1, Wf1, bf1, Wg2, bg2, Wf2, bf2,
           Wg3, bg3, Wf3, bf3, Wg4, bg4, Wf4, bf4,
           Ws, bs, Wt, bt, W1, b1, W2, b2):
    xs = (x1, x2, x3, x4)
    eis = (edge_index1, edge_index2, edge_index3, edge_index4)
    wgs = (Wg1, Wg2, Wg3, Wg4)
    bgs = (bg1, bg2, bg3, bg4)
    wfs = (Wf1, Wf2, Wf3, Wf4)
    bfs = (bf1, bf2, bf3, bf4)

    X = jnp.concatenate(
        [jnp.pad(x, ((0, NP - N_NODES), (0, 0))) for x in xs], axis=0)
    SRC = jnp.concatenate(
        [jnp.pad(ei[0], (0, EPG - ei.shape[1]), constant_values=NP - 1)
         for ei in eis])
    DST = jnp.concatenate(
        [jnp.pad(ei[1], (0, EPG - ei.shape[1]), constant_values=NP - 1)
         for ei in eis])
    WG = jnp.concatenate(
        [jnp.pad(w, ((0, 0), (0, 12))) for w in wgs], axis=0)
    BG = jnp.concatenate([jnp.pad(b, (0, 12)) for b in bgs])
    WFT = jnp.concatenate(
        [jnp.pad(w.T, ((0, 1), (0, 15))) for w in wfs], axis=0)
    BFL = jnp.pad(jnp.concatenate([jnp.pad(b, (0, 1)) for b in bfs]),
                  (0, 16))
    WSF = jnp.pad(Ws[:, 0], (0, 20))
    W1F = jnp.pad(W1[:, 0], (0, 23))
    TD = jnp.pad(t_data[0], (0, 8))
    WT = jnp.pad(Wt, ((0, 0), (0, 2)))
    BT = jnp.pad(bt, (0, 2))
    CST = jnp.concatenate(
        [bs, b1, b2, W2[:, 0], jnp.zeros((11,), F32)])

    mesh = plsc.VectorSubcoreMesh(
        core_axis_name="c", subcore_axis_name="s", num_cores=1)
    run = pl.kernel(
        _body,
        out_type=jax.ShapeDtypeStruct((16,), F32),
        mesh=mesh,
        scratch_types=[
            pltpu.VMEM_SHARED((NG * NP, 32), F32),
            pltpu.VMEM_SHARED((NG * NP, 32), F32),
            pltpu.VMEM_SHARED((NT, 32), F32),
            pltpu.VMEM((ROWS, 32), F32),
            pltpu.VMEM((32, 32), F32),
            pltpu.VMEM((EPT,), I32),
            pltpu.VMEM((EPG,), I32),
            pltpu.VMEM((2, 128), I32),
            pltpu.VMEM((2, 128), I32),
            pltpu.VMEM((ROWS, 32), F32),
            pltpu.VMEM((EPT, 32), F32),
            pltpu.VMEM((ROWS, 32), F32),
            pltpu.VMEM((EPG,), F32),
            pltpu.VMEM((512,), F32),
            pltpu.VMEM((FPT + 64,), F32),
            pltpu.VMEM((FPT + 64,), F32),
            pltpu.VMEM((8, 32), F32),
            pltpu.VMEM((48,), F32),
            pltpu.VMEM((80,), F32),
            pltpu.VMEM((32,), F32),
            pltpu.VMEM((32,), F32),
            pltpu.VMEM((24, 16), F32),
            pltpu.VMEM((16,), F32),
            pltpu.VMEM((48,), F32),
            pltpu.VMEM((16,), F32),
            pltpu.VMEM((1, 32), F32),
            pltpu.VMEM((NT, 32), F32),
            pltpu.VMEM((16,), F32),
            pltpu.SemaphoreType.DMA,
        ],
    )
    out16 = run(X, SRC, DST, WG, BG, WFT, BFL, WSF, W1F, TD, WT, BT, CST)
    return out16[0:1]
